# baseline (speedup 1.0000x reference)
import functools

import jax
import jax.numpy as jnp
from jax import lax
from jax.experimental import pallas as pl
from jax.experimental.pallas import tpu as pltpu
from jax.experimental.pallas import tpu_sc as plsc

_MAX_LEN = 4096


@functools.lru_cache(maxsize=None)
def _build(B, D, T, L):
    NCH = 4
    DC = D // NCH
    JC = 8
    JW = L // JC
    mesh = plsc.VectorSubcoreMesh(core_axis_name="c", subcore_axis_name="s")

    @functools.partial(
        pl.kernel,
        out_type=(
            jax.ShapeDtypeStruct((D + 2, B, L), jnp.float32),
            jax.ShapeDtypeStruct((B, 16), jnp.int32),
        ),
        mesh=mesh,
        compiler_params=pltpu.CompilerParams(needs_layout_passes=False),
        scratch_types=[
            pltpu.VMEM((DC, T), jnp.float32),
            pltpu.VMEM((T,), jnp.int32),
            pltpu.VMEM((T,), jnp.int32),
            pltpu.VMEM((L,), jnp.int32),
            pltpu.VMEM((DC, JW), jnp.float32),
            pltpu.VMEM((DC, JW), jnp.float32),
            pltpu.VMEM((2, L), jnp.float32),
            pltpu.VMEM((16,), jnp.int32),
            pltpu.SemaphoreType.DMA,
            pltpu.SemaphoreType.DMA,
            pltpu.SemaphoreType.DMA,
            pltpu.SemaphoreType.DMA,
        ],
    )
    def k(x_hbm, dur_hbm, out_hbm, mel_hbm,
          xb, db, sb, ib, ob0, ob1, wb, mb, sx, s0, s1, sw):
        wid = lax.axis_index("s") * 2 + lax.axis_index("c")
        b = wid // NCH
        ch = wid % NCH

        xh = pltpu.async_copy(x_hbm.at[b, pl.ds(ch * DC, DC), :], xb, sx)
        pltpu.sync_copy(dur_hbm.at[b], db)

        zeros16 = jnp.zeros((16,), jnp.int32)
        ones16 = jnp.ones((16,), jnp.int32)
        iota16 = lax.iota(jnp.int32, 16)
        fifteen = jnp.full((16,), 15, jnp.int32)

        def bcast_last(v):
            return v.at[fifteen].get(mode="promise_in_bounds")

        @plsc.parallel_loop(0, L // 16)
        def zero_body(i):
            ib[pl.ds(i * 16, 16)] = zeros16

        @plsc.parallel_loop(0, T // 16, carry=zeros16)
        def cum_body(i, carry):
            d = db[pl.ds(i * 16, 16)]
            de = jnp.maximum(jnp.minimum(jnp.abs(d), 10000), 1)
            db[pl.ds(i * 16, 16)] = de
            cm = plsc.cumsum(de) + carry
            sb[pl.ds(i * 16, 16)] = cm - de
            plsc.store_scatter(ib, [jnp.minimum(cm, L - 1)], ones16)
            return bcast_last(cm)

        totv = cum_body

        one_f = jnp.full((16,), 1.0, jnp.float32)
        zero_f = jnp.zeros((16,), jnp.float32)

        mb[...] = totv
        t = totv[0]
        nv = (t + 15) // 16

        @plsc.parallel_loop(0, nv, carry=zeros16)
        def idx_body(i, carry):
            v = ib[pl.ds(i * 16, 16)]
            cm = plsc.cumsum(v) + carry
            ib[pl.ds(i * 16, 16)] = jnp.minimum(cm, T - 1)
            return bcast_last(cm)

        xh.wait()

        out_rows = out_hbm.at[pl.ds(ch * DC, DC), b, pl.ds(0, JW)]
        bufs = (ob0, ob1)
        sems = (s0, s1)

        def jc_pair(jcp, _):
            for k2 in range(2):
                obk, sem = bufs[k2], sems[k2]
                jc = jcp * 2 + k2
                start = jc * JW

                @pl.when(jcp > 0)
                def _():
                    pltpu.make_async_copy(obk, out_rows, sem).wait()

                @pl.when(start < t)
                def _():
                    @plsc.parallel_loop(0, JW // 16)
                    def gather_body(jv, obk=obk):
                        base = start + jv * 16
                        idxv = ib[pl.ds(base, 16)]
                        mk = jnp.where(base + iota16 < totv, one_f, zero_f)
                        for dd in range(DC):
                            rowv = jnp.full((16,), dd, jnp.int32)
                            g = plsc.load_gather(xb, [rowv, idxv])
                            obk[dd, pl.ds(jv * 16, 16)] = g * mk

                @pl.when(jnp.logical_and(start >= t, (jc - 2) * JW < t))
                def _():
                    def zfill_body(jv, c, obk=obk):
                        for dd in range(DC):
                            obk[dd, pl.ds(jv * 16, 16)] = zero_f
                        return c

                    lax.fori_loop(0, JW // 16, zfill_body, 0)

                col = pl.multiple_of(start, JW)
                pltpu.async_copy(
                    obk, out_hbm.at[pl.ds(ch * DC, DC), b, pl.ds(col, JW)],
                    sem,
                )

            @pl.when(ch == NCH - 1)
            def _():
                NW = L // 16 // (JC // 2)

                @plsc.parallel_loop(jcp * NW, (jcp + 1) * NW)
                def wl_body(jv):
                    base = jv * 16
                    idxv = ib[pl.ds(base, 16)]
                    s = plsc.load_gather(sb, [idxv])
                    le = plsc.load_gather(db, [idxv])
                    jvec = base + iota16
                    mk = jnp.where(jvec < totv, one_f, zero_f)
                    wb[0, pl.ds(base, 16)] = (
                        (jvec - s).astype(jnp.float32) * mk)
                    wb[1, pl.ds(base, 16)] = le.astype(jnp.float32) * mk

            return 0

        lax.fori_loop(0, JC // 2, jc_pair, 0)
        pltpu.make_async_copy(bufs[0], out_rows, s0).wait()
        pltpu.make_async_copy(bufs[1], out_rows, s1).wait()

        @pl.when(ch == NCH - 1)
        def _():
            pltpu.async_copy(wb, out_hbm.at[pl.ds(D, 2), b, :], sw).wait()

        @pl.when(ch == 0)
        def _():
            pltpu.sync_copy(mb, mel_hbm.at[b])

    return k


def kernel(x, duration, max_len):
    B, D, T = x.shape
    k = _build(B, D, T, _MAX_LEN)
    out, mel = k(x, duration)
    return jnp.transpose(out, (1, 0, 2)), mel[:, 0]


# speedup vs baseline: 1.0242x; 1.0242x over previous
import functools

import jax
import jax.numpy as jnp
from jax import lax
from jax.experimental import pallas as pl
from jax.experimental.pallas import tpu as pltpu
from jax.experimental.pallas import tpu_sc as plsc

_MAX_LEN = 4096


@functools.lru_cache(maxsize=None)
def _build(B, D, T, L):
    NCH = 4
    DC = D // NCH
    JC = 8
    JW = L // JC
    mesh = plsc.VectorSubcoreMesh(core_axis_name="c", subcore_axis_name="s")

    @functools.partial(
        pl.kernel,
        out_type=(
            jax.ShapeDtypeStruct((D + 2, B, L), jnp.float32),
            jax.ShapeDtypeStruct((B, 16), jnp.int32),
        ),
        mesh=mesh,
        compiler_params=pltpu.CompilerParams(needs_layout_passes=False),
        scratch_types=[
            pltpu.VMEM((DC, T), jnp.float32),
            pltpu.VMEM((T,), jnp.int32),
            pltpu.VMEM((T,), jnp.int32),
            pltpu.VMEM((L,), jnp.int32),
            pltpu.VMEM((DC, JW), jnp.float32),
            pltpu.VMEM((DC, JW), jnp.float32),
            pltpu.VMEM((2, L), jnp.float32),
            pltpu.VMEM((16,), jnp.int32),
            pltpu.SemaphoreType.DMA,
            pltpu.SemaphoreType.DMA,
            pltpu.SemaphoreType.DMA,
            pltpu.SemaphoreType.DMA,
        ],
    )
    def k(x_hbm, dur_hbm, out_hbm, mel_hbm,
          xb, db, sb, ib, ob0, ob1, wb, mb, sx, s0, s1, sw):
        wid = lax.axis_index("s") * 2 + lax.axis_index("c")
        b = wid // NCH
        ch = wid % NCH

        xh = pltpu.async_copy(x_hbm.at[b, pl.ds(ch * DC, DC), :], xb, sx)
        pltpu.sync_copy(dur_hbm.at[b], db)

        zeros16 = jnp.zeros((16,), jnp.int32)
        ones16 = jnp.ones((16,), jnp.int32)
        iota16 = lax.iota(jnp.int32, 16)
        fifteen = jnp.full((16,), 15, jnp.int32)

        def bcast_last(v):
            return v.at[fifteen].get(mode="promise_in_bounds")

        @plsc.parallel_loop(0, L // 16)
        def zero_body(i):
            ib[pl.ds(i * 16, 16)] = zeros16

        @plsc.parallel_loop(0, T // 16, carry=zeros16)
        def cum_body(i, carry):
            d = db[pl.ds(i * 16, 16)]
            de = jnp.maximum(jnp.minimum(jnp.abs(d), 10000), 1)
            db[pl.ds(i * 16, 16)] = de
            cm = plsc.cumsum(de) + carry
            sb[pl.ds(i * 16, 16)] = cm - de
            plsc.store_scatter(ib, [jnp.minimum(cm, L - 1)], ones16)
            return bcast_last(cm)

        totv = cum_body

        one_f = jnp.full((16,), 1.0, jnp.float32)
        zero_f = jnp.zeros((16,), jnp.float32)

        mb[...] = totv
        t = totv[0]
        nv = (t + 15) // 16

        @plsc.parallel_loop(0, nv, carry=zeros16)
        def idx_body(i, carry):
            v = ib[pl.ds(i * 16, 16)]
            cm = plsc.cumsum(v) + carry
            ib[pl.ds(i * 16, 16)] = jnp.minimum(cm, T - 1)
            return bcast_last(cm)

        xh.wait()

        out_rows = out_hbm.at[pl.ds(ch * DC, DC), b, pl.ds(0, JW)]
        bufs = (ob0, ob1)
        sems = (s0, s1)

        def jc_pair(jcp, _):
            for k2 in range(2):
                obk, sem = bufs[k2], sems[k2]
                jc = jcp * 2 + k2
                start = jc * JW

                @pl.when(jcp > 0)
                def _():
                    pltpu.make_async_copy(obk, out_rows, sem).wait()

                @pl.when(start < t)
                def _():
                    @plsc.parallel_loop(0, JW // 16)
                    def gather_body(jv, obk=obk):
                        base = start + jv * 16
                        idxv = ib[pl.ds(base, 16)]
                        mk = jnp.where(base + iota16 < totv, one_f, zero_f)
                        for dd in range(DC):
                            rowv = jnp.full((16,), dd, jnp.int32)
                            g = plsc.load_gather(xb, [rowv, idxv])
                            obk[dd, pl.ds(jv * 16, 16)] = g * mk

                @pl.when(jnp.logical_and(start >= t, (jc - 2) * JW < t))
                def _():
                    def zfill_body(jv, c, obk=obk):
                        for dd in range(DC):
                            obk[dd, pl.ds(jv * 16, 16)] = zero_f
                        return c

                    lax.fori_loop(0, JW // 16, zfill_body, 0)

                col = pl.multiple_of(start, JW)
                pltpu.async_copy(
                    obk, out_hbm.at[pl.ds(ch * DC, DC), b, pl.ds(col, JW)],
                    sem,
                )
            return 0

        lax.fori_loop(0, JC // 2, jc_pair, 0)
        pltpu.make_async_copy(bufs[0], out_rows, s0).wait()
        pltpu.make_async_copy(bufs[1], out_rows, s1).wait()

        @pl.when(ch == NCH - 1)
        def _():
            @plsc.parallel_loop(0, nv, unroll=2)
            def wl_body(jv):
                base = jv * 16
                idxv = ib[pl.ds(base, 16)]
                s = plsc.load_gather(sb, [idxv])
                le = plsc.load_gather(db, [idxv])
                jvec = base + iota16
                mk = jnp.where(jvec < totv, one_f, zero_f)
                wb[0, pl.ds(base, 16)] = (jvec - s).astype(jnp.float32) * mk
                wb[1, pl.ds(base, 16)] = le.astype(jnp.float32) * mk

            @plsc.parallel_loop(nv, L // 16, unroll=2)
            def wl_zero(jv):
                wb[0, pl.ds(jv * 16, 16)] = zero_f
                wb[1, pl.ds(jv * 16, 16)] = zero_f

            pltpu.async_copy(wb, out_hbm.at[pl.ds(D, 2), b, :], sw).wait()

        @pl.when(ch == 0)
        def _():
            pltpu.sync_copy(mb, mel_hbm.at[b])

    return k


def kernel(x, duration, max_len):
    B, D, T = x.shape
    k = _build(B, D, T, _MAX_LEN)
    out, mel = k(x, duration)
    return jnp.transpose(out, (1, 0, 2)), mel[:, 0]
